# baseline (speedup 1.0000x reference)
import jax
import jax.numpy as jnp
from jax import lax
from jax.experimental import pallas as pl
from jax.experimental.pallas import tpu as pltpu
from jax.experimental.pallas import tpu_sc as plsc

_S = 13
_CELLS = _S * _S
_NCH = 19
_CONF2 = 36
_T = 20
_L = 16
_NW = 32
_BATCH = 128
_BPW = _BATCH // _NW


def _body(preds_hbm, c2_hbm, tg_hbm, out_hbm, tg_v, blk_v, c2_v, cellbuf, acc_v):
    wid = lax.axis_index("s") * 2 + lax.axis_index("c")
    lanes = lax.iota(jnp.int32, _L)
    zeros = jnp.zeros((_L,), jnp.float32)

    def splat(v):
        return jnp.full((_L,), v, jnp.int32)

    b0 = wid * _BPW
    pltpu.sync_copy(preds_hbm.at[pl.ds(b0, _BPW)], blk_v)
    pltpu.sync_copy(c2_hbm.at[pl.ds(b0, _BPW)], c2_v)
    pltpu.sync_copy(tg_hbm.at[pl.ds(b0, _BPW)], tg_v)

    acc_m = zeros
    acc_c = zeros

    for i in range(_BPW):
        isp = splat(i)

        def pick(ch, gy, gx):
            return plsc.load_gather(blk_v, [isp, splat(ch), gy, gx])

        per_chunk = []
        for chunk in range(2):
            tvec = lanes + chunk * _L
            fidx = jnp.minimum(tvec, _T - 1) * 5

            def field(f):
                return plsc.load_gather(tg_v, [isp, fidx + f])

            cls = field(0)
            cx = field(1)
            cy = field(2)
            w = field(3)
            h = field(4)

            gx = (cx * _S).astype(jnp.int32)
            gy = (cy * _S).astype(jnp.int32)
            valid = (gx < _S) & (gy < _S) & (tvec < _T)
            gxc = jnp.clip(gx, 0, _S - 1)
            gyc = jnp.clip(gy, 0, _S - 1)

            d1 = pick(1, gyc, gxc) - cx
            d2 = pick(2, gyc, gxc) - cy
            d3 = pick(3, gyc, gxc) - w
            d4 = pick(4, gyc, gxc) - h
            coord = d1 * d1 + d2 * d2 + d3 * d3 + d4 * d4

            k = cls.astype(jnp.int32)
            cls_l = zeros
            for c in range(13):
                p = pick(5 + c, gyc, gxc)
                d = jnp.where(k == c, p - 1.0, p)
                cls_l = cls_l + d * d

            contrib = 5.0 * coord + cls_l
            acc_m = acc_m + jnp.where(valid, contrib, 0.0)

            plsc.store_scatter(cellbuf, [gyc, gxc], tvec, mask=valid)
            per_chunk.append((tvec, gyc, gxc, valid))

        for tvec, gyc, gxc, valid in per_chunk:
            rb = plsc.load_gather(cellbuf, [gyc, gxc])
            winner = valid & (rb == tvec)
            c0t = pick(0, gyc, gxc)
            acc_c = acc_c - jnp.where(winner, c0t * c0t, 0.0)

        for j in range(11):
            cellv = lanes + j * _L
            live = cellv < _CELLS
            cellc = jnp.minimum(cellv, _CELLS - 1)
            yv = cellc // _S
            xv = cellc - yv * _S
            v0 = pick(0, yv, xv)
            v1 = pick(_NCH - 1, yv, xv)
            v2 = plsc.load_gather(c2_v, [isp, yv, xv])
            ssq = v0 * v0 + v1 * v1 + v2 * v2
            if j == 10:
                ssq = jnp.where(live, ssq, 0.0)
            acc_c = acc_c + ssq

    acc_v[...] = acc_m + 0.5 * acc_c
    pltpu.sync_copy(acc_v, out_hbm.at[wid])


def kernel(predictions, targets):
    preds19 = predictions[:, :_NCH]
    conf2 = predictions[:, _CONF2]
    tg2 = targets.reshape(_BATCH, 5 * _T)
    mesh = plsc.VectorSubcoreMesh(
        core_axis_name="c", subcore_axis_name="s", num_cores=2, num_subcores=16)
    out = pl.kernel(
        _body,
        out_type=jax.ShapeDtypeStruct((_NW, _L), jnp.float32),
        mesh=mesh,
        compiler_params=pltpu.CompilerParams(
            use_tc_tiling_on_sc=False, needs_layout_passes=False),
        scratch_types=[
            pltpu.VMEM((_BPW, 5 * _T), jnp.float32),
            pltpu.VMEM((_BPW, _NCH, _S, _S), jnp.float32),
            pltpu.VMEM((_BPW, _S, _S), jnp.float32),
            pltpu.VMEM((_S, _S), jnp.int32),
            pltpu.VMEM((_L,), jnp.float32),
        ],
    )(preds19, conf2, tg2)
    return jnp.sum(out) / _BATCH


# speedup vs baseline: 1.8053x; 1.8053x over previous
import jax
import jax.numpy as jnp
from jax import lax
from jax.experimental import pallas as pl
from jax.experimental.pallas import tpu as pltpu
from jax.experimental.pallas import tpu_sc as plsc

_S = 13
_CELLS = _S * _S
_NCH = 20
_T = 20
_L = 16
_NW = 32
_BATCH = 128
_BPW = _BATCH // _NW


def _body(preds_hbm, tg_hbm, out_hbm, tg_v, blk_v, cellbuf, acc_v):
    wid = lax.axis_index("s") * 2 + lax.axis_index("c")
    lanes = lax.iota(jnp.int32, _L)
    zeros = jnp.zeros((_L,), jnp.float32)
    tail9 = lanes < (_CELLS - 10 * _L)
    tail_idx = jnp.minimum(lanes + 10 * _L, _CELLS - 1)

    def splat(v):
        return jnp.full((_L,), v, jnp.int32)

    b0 = wid * _BPW
    pltpu.sync_copy(preds_hbm.at[pl.ds(b0, _BPW)], blk_v)
    pltpu.sync_copy(tg_hbm.at[pl.ds(b0, _BPW)], tg_v)

    acc_m = zeros
    acc_c = zeros

    for i in range(_BPW):
        isp = splat(i)

        def pick(ch, cell):
            return plsc.load_gather(blk_v, [isp, splat(ch), cell])

        per_chunk = []
        for chunk in range(2):
            tvec = lanes + chunk * _L
            fidx = jnp.minimum(tvec, _T - 1) * 5

            def field(f):
                return plsc.load_gather(tg_v, [isp, fidx + f])

            cls = field(0)
            cx = field(1)
            cy = field(2)
            w = field(3)
            h = field(4)

            gx = (cx * _S).astype(jnp.int32)
            gy = (cy * _S).astype(jnp.int32)
            valid = (gx < _S) & (gy < _S) & (tvec < _T)
            gxc = jnp.clip(gx, 0, _S - 1)
            gyc = jnp.clip(gy, 0, _S - 1)
            cell = gyc * _S + gxc

            d1 = pick(1, cell) - cx
            d2 = pick(2, cell) - cy
            d3 = pick(3, cell) - w
            d4 = pick(4, cell) - h
            coord = d1 * d1 + d2 * d2 + d3 * d3 + d4 * d4

            k = cls.astype(jnp.int32)
            cls_l = zeros
            for c in range(13):
                p = pick(5 + c, cell)
                d = jnp.where(k == c, p - 1.0, p)
                cls_l = cls_l + d * d

            contrib = 5.0 * coord + cls_l
            acc_m = acc_m + jnp.where(valid, contrib, 0.0)

            plsc.store_scatter(cellbuf, [cell], tvec, mask=valid)
            per_chunk.append((tvec, cell, valid))

        for tvec, cell, valid in per_chunk:
            rb = plsc.load_gather(cellbuf, [cell])
            winner = valid & (rb == tvec)
            c0t = pick(0, cell)
            acc_c = acc_c - jnp.where(winner, c0t * c0t, 0.0)

        for ch in (0, _NCH - 2, _NCH - 1):
            for j in range(10):
                v = blk_v[i, ch, pl.ds(j * _L, _L)]
                acc_c = acc_c + v * v
            v = pick(ch, tail_idx)
            v = jnp.where(tail9, v, 0.0)
            acc_c = acc_c + v * v

    acc_v[...] = acc_m + 0.5 * acc_c
    pltpu.sync_copy(acc_v, out_hbm.at[wid])


_CH20 = tuple(range(_NCH - 1)) + (36,)


def kernel(predictions, targets):
    preds20 = jnp.take(
        predictions, jnp.array(_CH20, jnp.int32), axis=1
    ).reshape(_BATCH, _NCH, _CELLS)
    tg2 = targets.reshape(_BATCH, 5 * _T)
    mesh = plsc.VectorSubcoreMesh(
        core_axis_name="c", subcore_axis_name="s", num_cores=2, num_subcores=16)
    out = pl.kernel(
        _body,
        out_type=jax.ShapeDtypeStruct((_NW, _L), jnp.float32),
        mesh=mesh,
        compiler_params=pltpu.CompilerParams(
            use_tc_tiling_on_sc=False, needs_layout_passes=False),
        scratch_types=[
            pltpu.VMEM((_BPW, 5 * _T), jnp.float32),
            pltpu.VMEM((_BPW, _NCH, _CELLS), jnp.float32),
            pltpu.VMEM((_CELLS,), jnp.int32),
            pltpu.VMEM((_L,), jnp.float32),
        ],
    )(preds20, tg2)
    return jnp.sum(out) / _BATCH


# speedup vs baseline: 2.2824x; 1.2643x over previous
import jax
import jax.numpy as jnp
from jax import lax
from jax.experimental import pallas as pl
from jax.experimental.pallas import tpu as pltpu
from jax.experimental.pallas import tpu_sc as plsc

_S = 13
_CELLS = _S * _S
_CP = 176
_NCH = 20
_T = 20
_TP = 104
_L = 16
_NW = 32
_BATCH = 128
_BPW = _BATCH // _NW


def _body(preds_hbm, tg_hbm, out_hbm, tg_v, blk_v, cellbuf, acc_v):
    wid = lax.axis_index("s") * 2 + lax.axis_index("c")
    lanes = lax.iota(jnp.int32, _L)
    zeros = jnp.zeros((_L,), jnp.float32)

    def splat(v):
        return jnp.full((_L,), v, jnp.int32)

    b0 = wid * _BPW
    pltpu.sync_copy(preds_hbm.at[pl.ds(b0, _BPW)], blk_v)
    pltpu.sync_copy(tg_hbm.at[pl.ds(b0, _BPW)], tg_v)

    acc_m = zeros
    acc_c = zeros

    for i in range(_BPW):
        isp = splat(i)

        def pick(ch, cell):
            return plsc.load_gather(blk_v, [isp, splat(ch), cell])

        per_chunk = []
        for chunk in range(2):
            tvec = lanes + chunk * _L
            fidx = jnp.minimum(tvec, _T - 1) * 5

            def field(f):
                return plsc.load_gather(tg_v, [isp, fidx + f])

            cls = field(0)
            cx = field(1)
            cy = field(2)
            w = field(3)
            h = field(4)

            gx = (cx * _S).astype(jnp.int32)
            gy = (cy * _S).astype(jnp.int32)
            valid = (gx < _S) & (gy < _S) & (tvec < _T)
            gxc = jnp.clip(gx, 0, _S - 1)
            gyc = jnp.clip(gy, 0, _S - 1)
            cell = gyc * _S + gxc

            d1 = pick(1, cell) - cx
            d2 = pick(2, cell) - cy
            d3 = pick(3, cell) - w
            d4 = pick(4, cell) - h
            coord = d1 * d1 + d2 * d2 + d3 * d3 + d4 * d4

            k = cls.astype(jnp.int32)
            cls_l = zeros
            for c in range(13):
                p = pick(5 + c, cell)
                d = jnp.where(k == c, p - 1.0, p)
                cls_l = cls_l + d * d

            contrib = 5.0 * coord + cls_l
            acc_m = acc_m + jnp.where(valid, contrib, 0.0)

            plsc.store_scatter(cellbuf, [cell], tvec, mask=valid)
            per_chunk.append((tvec, cell, valid))

        for tvec, cell, valid in per_chunk:
            rb = plsc.load_gather(cellbuf, [cell])
            winner = valid & (rb == tvec)
            c0t = pick(0, cell)
            acc_c = acc_c - jnp.where(winner, c0t * c0t, 0.0)

        for ch in (0, _NCH - 2, _NCH - 1):
            for j in range(_CP // _L):
                v = blk_v[i, ch, pl.ds(j * _L, _L)]
                acc_c = acc_c + v * v

    acc_v[...] = acc_m + 0.5 * acc_c
    pltpu.sync_copy(acc_v, out_hbm.at[wid])


def kernel(predictions, targets):
    preds20 = jnp.concatenate(
        [predictions[:, :_NCH - 1], predictions[:, 36:37]], axis=1
    ).reshape(_BATCH, _NCH, _CELLS)
    preds20 = jnp.pad(preds20, ((0, 0), (0, 0), (0, _CP - _CELLS)))
    tg2 = jnp.pad(targets.reshape(_BATCH, 5 * _T), ((0, 0), (0, _TP - 5 * _T)))
    mesh = plsc.VectorSubcoreMesh(
        core_axis_name="c", subcore_axis_name="s", num_cores=2, num_subcores=16)
    out = pl.kernel(
        _body,
        out_type=jax.ShapeDtypeStruct((_NW, _L), jnp.float32),
        mesh=mesh,
        compiler_params=pltpu.CompilerParams(
            use_tc_tiling_on_sc=False, needs_layout_passes=False),
        scratch_types=[
            pltpu.VMEM((_BPW, _TP), jnp.float32),
            pltpu.VMEM((_BPW, _NCH, _CP), jnp.float32),
            pltpu.VMEM((_CELLS,), jnp.int32),
            pltpu.VMEM((_L,), jnp.float32),
        ],
    )(preds20, tg2)
    return jnp.sum(out) / _BATCH


# speedup vs baseline: 2.6006x; 1.1394x over previous
import jax
import jax.numpy as jnp
from jax import lax
from jax.experimental import pallas as pl
from jax.experimental.pallas import tpu as pltpu
from jax.experimental.pallas import tpu_sc as plsc

_S = 13
_CELLS = _S * _S
_T = 20
_TP = 104
_L = 16
_NW = 32
_BATCH = 128
_BPW = _BATCH // _NW
_CONF1 = 18
_CONF2 = 23


def _body(preds_hbm, tg_hbm, out_hbm, tg_v, blk_v, cellbuf, acc_v):
    wid = lax.axis_index("s") * 2 + lax.axis_index("c")
    lanes = lax.iota(jnp.int32, _L)
    zeros = jnp.zeros((_L,), jnp.float32)

    def splat(v):
        return jnp.full((_L,), v, jnp.int32)

    b0 = wid * _BPW
    pltpu.sync_copy(preds_hbm.at[pl.ds(b0, _BPW)], blk_v)
    pltpu.sync_copy(tg_hbm.at[pl.ds(b0, _BPW)], tg_v)

    acc_m = zeros
    acc_c = zeros

    for i in range(_BPW):
        isp = splat(i)

        def pick(ch, hv, jv):
            return plsc.load_gather(
                blk_v, [isp, splat(ch // 8), hv, splat(ch % 8), jv])

        per_chunk = []
        for chunk in range(2):
            tvec = lanes + chunk * _L
            fidx = jnp.minimum(tvec, _T - 1) * 5

            def field(f):
                return plsc.load_gather(tg_v, [isp, fidx + f])

            cls = field(0)
            cx = field(1)
            cy = field(2)
            w = field(3)
            h = field(4)

            gx = (cx * _S).astype(jnp.int32)
            gy = (cy * _S).astype(jnp.int32)
            valid = (gx < _S) & (gy < _S) & (tvec < _T)
            gxc = jnp.clip(gx, 0, _S - 1)
            gyc = jnp.clip(gy, 0, _S - 1)
            cell = gyc * _S + gxc
            hv = lax.shift_right_logical(cell, 7)
            jv = cell & 127

            d1 = pick(1, hv, jv) - cx
            d2 = pick(2, hv, jv) - cy
            d3 = pick(3, hv, jv) - w
            d4 = pick(4, hv, jv) - h
            coord = d1 * d1 + d2 * d2 + d3 * d3 + d4 * d4

            k = cls.astype(jnp.int32)
            cls_l = zeros
            for c in range(13):
                p = pick(5 + c, hv, jv)
                d = jnp.where(k == c, p - 1.0, p)
                cls_l = cls_l + d * d

            contrib = 5.0 * coord + cls_l
            acc_m = acc_m + jnp.where(valid, contrib, 0.0)

            plsc.store_scatter(cellbuf, [cell], tvec, mask=valid)
            per_chunk.append((tvec, cell, hv, jv, valid))

        for tvec, cell, hv, jv, valid in per_chunk:
            rb = plsc.load_gather(cellbuf, [cell])
            winner = valid & (rb == tvec)
            c0t = pick(0, hv, jv)
            acc_c = acc_c - jnp.where(winner, c0t * c0t, 0.0)

        for ch in (0, _CONF1, _CONF2):
            for hh in range(2):
                for jj in range(8):
                    v = blk_v[i, ch // 8, hh, ch % 8, pl.ds(jj * _L, _L)]
                    acc_c = acc_c + v * v

    acc_v[...] = acc_m + 0.5 * acc_c
    pltpu.sync_copy(acc_v, out_hbm.at[wid])


def kernel(predictions, targets):
    p24 = jnp.concatenate(
        [predictions[:, :23], predictions[:, 36:37]], axis=1
    ).reshape(_BATCH, 24, _CELLS)
    p24 = jnp.pad(p24, ((0, 0), (0, 0), (0, 256 - _CELLS)))
    p24 = p24.reshape(_BATCH, 3, 8, 2, 128).transpose(0, 1, 3, 2, 4)
    tg2 = jnp.pad(targets.reshape(_BATCH, 5 * _T), ((0, 0), (0, _TP - 5 * _T)))
    mesh = plsc.VectorSubcoreMesh(
        core_axis_name="c", subcore_axis_name="s", num_cores=2, num_subcores=16)
    out = pl.kernel(
        _body,
        out_type=jax.ShapeDtypeStruct((_NW, _L), jnp.float32),
        mesh=mesh,
        compiler_params=pltpu.CompilerParams(
            use_tc_tiling_on_sc=False, needs_layout_passes=False),
        scratch_types=[
            pltpu.VMEM((_BPW, _TP), jnp.float32),
            pltpu.VMEM((_BPW, 3, 2, 8, 128), jnp.float32),
            pltpu.VMEM((_CELLS,), jnp.int32),
            pltpu.VMEM((_L,), jnp.float32),
        ],
    )(p24, tg2)
    return jnp.sum(out) / _BATCH
